# baseline (speedup 1.0000x reference)
import jax
import jax.numpy as jnp
from jax import lax
from jax.experimental import pallas as pl
from jax.experimental.pallas import tpu as pltpu
from jax.experimental.pallas import tpu_sc as plsc

_D = 128
_C = 128
_G = _C // 16
_NW = 32


def _distmult_body(src_hbm, dst_hbm, et_hbm, z_hbm, rel_hbm, out_hbm,
                   rel_v, sidx, didx, tidx, srows, drows, outv,
                   sem_s, sem_d):
    wid = lax.axis_index("s") * 2 + lax.axis_index("c")
    e_per = src_hbm.shape[0] // _NW
    n_chunks = (e_per + _C - 1) // _C
    last_base = e_per - _C
    e_base = wid * e_per

    pltpu.sync_copy(rel_hbm, rel_v)

    row_idx = [lax.iota(jnp.int32, 16) + (g * 16) for g in range(_G)]

    def chunk_body(c, carry):
        base = e_base + jnp.minimum(c * _C, last_base)
        pltpu.sync_copy(src_hbm.at[pl.ds(base, _C)], sidx)
        pltpu.sync_copy(dst_hbm.at[pl.ds(base, _C)], didx)
        pltpu.sync_copy(et_hbm.at[pl.ds(base, _C)], tidx)
        cp_s = pltpu.async_copy(z_hbm.at[sidx], srows, sem_s)
        cp_d = pltpu.async_copy(z_hbm.at[didx], drows, sem_d)
        cp_s.wait()
        cp_d.wait()
        tvs = [tidx[pl.ds(g * 16, 16)] for g in range(_G)]

        def d_body(d, accs):
            col = jnp.full((16,), d, jnp.int32)
            new = []
            for g in range(_G):
                s = plsc.load_gather(srows, [row_idx[g], col])
                r = plsc.load_gather(rel_v, [tvs[g], col])
                t = plsc.load_gather(drows, [row_idx[g], col])
                new.append(accs[g] + s * r * t)
            return tuple(new)

        accs = lax.fori_loop(
            0, _D, d_body,
            tuple(jnp.zeros((16,), jnp.float32) for _ in range(_G)))
        for g in range(_G):
            outv[pl.ds(g * 16, 16)] = accs[g]
        pltpu.sync_copy(outv, out_hbm.at[pl.ds(base, _C)])
        return carry

    lax.fori_loop(0, n_chunks, chunk_body, 0)


def kernel(z, edge_index, edge_type, rel_emb):
    src = edge_index[0].astype(jnp.int32)
    dst = edge_index[1].astype(jnp.int32)
    et = edge_type.astype(jnp.int32)
    e = src.shape[0]
    mesh = plsc.VectorSubcoreMesh(core_axis_name="c", subcore_axis_name="s")
    f = pl.kernel(
        _distmult_body,
        out_type=jax.ShapeDtypeStruct((e,), jnp.float32),
        mesh=mesh,
        compiler_params=pltpu.CompilerParams(needs_layout_passes=False),
        scratch_types=[
            pltpu.VMEM(rel_emb.shape, jnp.float32),
            pltpu.VMEM((_C,), jnp.int32),
            pltpu.VMEM((_C,), jnp.int32),
            pltpu.VMEM((_C,), jnp.int32),
            pltpu.VMEM((_C, _D), jnp.float32),
            pltpu.VMEM((_C, _D), jnp.float32),
            pltpu.VMEM((_C,), jnp.float32),
            pltpu.SemaphoreType.DMA,
            pltpu.SemaphoreType.DMA,
        ],
    )
    return f(src, dst, et, z, rel_emb)


# speedup vs baseline: 5.5901x; 5.5901x over previous
import jax
import jax.numpy as jnp
from jax import lax
from jax.experimental import pallas as pl
from jax.experimental.pallas import tpu as pltpu
from jax.experimental.pallas import tpu_sc as plsc

_D = 128
_C = 128
_G = _C // 16
_NW = 32


def _distmult_body(src_hbm, dst_hbm, et_hbm, z_hbm, rel_hbm, out_hbm,
                   rel_v, sidx, didx, tidx, srows, drows, outv,
                   sem_s, sem_d):
    wid = lax.axis_index("s") * 2 + lax.axis_index("c")
    e_per = src_hbm.shape[0] // _NW
    n_chunks = (e_per + _C - 1) // _C
    last_base = e_per - _C
    e_base = wid * e_per

    pltpu.sync_copy(rel_hbm, rel_v)

    row_idx = [lax.iota(jnp.int32, 16) + (g * 16) for g in range(_G)]

    def chunk_body(c, carry):
        base = e_base + jnp.minimum(c * _C, last_base)
        pltpu.sync_copy(src_hbm.at[pl.ds(base, _C)], sidx)
        pltpu.sync_copy(dst_hbm.at[pl.ds(base, _C)], didx)
        pltpu.sync_copy(et_hbm.at[pl.ds(base, _C)], tidx)
        cp_s = pltpu.async_copy(z_hbm.at[sidx], srows, sem_s)
        cp_d = pltpu.async_copy(z_hbm.at[didx], drows, sem_d)
        cp_s.wait()
        cp_d.wait()
        tvs = [tidx[pl.ds(g * 16, 16)] for g in range(_G)]

        lane = lax.iota(jnp.int32, 16)

        def d_body(d, accs):
            col = (jnp.full((16,), d, jnp.int32) + lane) & (_D - 1)
            new = []
            for g in range(_G):
                s = plsc.load_gather(srows, [row_idx[g], col])
                r = plsc.load_gather(rel_v, [tvs[g], col])
                t = plsc.load_gather(drows, [row_idx[g], col])
                new.append(accs[g] + s * r * t)
            return tuple(new)

        accs = lax.fori_loop(
            0, _D, d_body,
            tuple(jnp.zeros((16,), jnp.float32) for _ in range(_G)))
        for g in range(_G):
            outv[pl.ds(g * 16, 16)] = accs[g]
        pltpu.sync_copy(outv, out_hbm.at[pl.ds(base, _C)])
        return carry

    lax.fori_loop(0, n_chunks, chunk_body, 0)


def kernel(z, edge_index, edge_type, rel_emb):
    src = edge_index[0].astype(jnp.int32)
    dst = edge_index[1].astype(jnp.int32)
    et = edge_type.astype(jnp.int32)
    e = src.shape[0]
    mesh = plsc.VectorSubcoreMesh(core_axis_name="c", subcore_axis_name="s")
    f = pl.kernel(
        _distmult_body,
        out_type=jax.ShapeDtypeStruct((e,), jnp.float32),
        mesh=mesh,
        compiler_params=pltpu.CompilerParams(needs_layout_passes=False),
        scratch_types=[
            pltpu.VMEM(rel_emb.shape, jnp.float32),
            pltpu.VMEM((_C,), jnp.int32),
            pltpu.VMEM((_C,), jnp.int32),
            pltpu.VMEM((_C,), jnp.int32),
            pltpu.VMEM((_C, _D), jnp.float32),
            pltpu.VMEM((_C, _D), jnp.float32),
            pltpu.VMEM((_C,), jnp.float32),
            pltpu.SemaphoreType.DMA,
            pltpu.SemaphoreType.DMA,
        ],
    )
    return f(src, dst, et, z, rel_emb)


# speedup vs baseline: 12.2580x; 2.1928x over previous
import jax
import jax.numpy as jnp
from jax import lax
from jax.experimental import pallas as pl
from jax.experimental.pallas import tpu as pltpu
from jax.experimental.pallas import tpu_sc as plsc

_D = 128
_C = 128
_G = _C // 16
_NW = 32


def _distmult_body(src_hbm, dst_hbm, et_hbm, z_hbm, rel_hbm, out_hbm,
                   rel_v, src_v, dst_v, et_v, srows0, drows0, srows1, drows1,
                   outv, sem_s0, sem_d0, sem_s1, sem_d1):
    wid = lax.axis_index("s") * 2 + lax.axis_index("c")
    e_per = src_hbm.shape[0] // _NW
    n_chunks = -(-e_per // _C) + (-(-e_per // _C)) % 2
    last_base = e_per - _C
    e_base = wid * e_per

    pltpu.sync_copy(rel_hbm, rel_v)
    pltpu.sync_copy(src_hbm.at[pl.ds(e_base, e_per)], src_v)
    pltpu.sync_copy(dst_hbm.at[pl.ds(e_base, e_per)], dst_v)
    pltpu.sync_copy(et_hbm.at[pl.ds(e_base, e_per)], et_v)

    lane = lax.iota(jnp.int32, 16)
    row_idx = [lane + (g * 16) for g in range(_G)]

    def lbase_of(c):
        return jnp.minimum(c * _C, last_base)

    def issue(c, srows, drows, sem_s, sem_d):
        lbase = lbase_of(c)
        cs = pltpu.async_copy(
            z_hbm.at[src_v.at[pl.ds(lbase, _C)]], srows, sem_s)
        cd = pltpu.async_copy(
            z_hbm.at[dst_v.at[pl.ds(lbase, _C)]], drows, sem_d)
        return cs, cd

    def compute(c, srows, drows, sem_s, sem_d):
        lbase = lbase_of(c)
        pltpu.make_async_copy(z_hbm.at[src_v.at[pl.ds(lbase, _C)]],
                              srows, sem_s).wait()
        pltpu.make_async_copy(z_hbm.at[dst_v.at[pl.ds(lbase, _C)]],
                              drows, sem_d).wait()
        tvs = [et_v[pl.ds(lbase + g * 16, 16)] for g in range(_G)]

        def d_body(d, accs):
            col = (jnp.full((16,), d, jnp.int32) + lane) & (_D - 1)
            new = []
            for g in range(_G):
                s = plsc.load_gather(srows, [row_idx[g], col])
                r = plsc.load_gather(rel_v, [tvs[g], col])
                t = plsc.load_gather(drows, [row_idx[g], col])
                new.append(accs[g] + s * r * t)
            return tuple(new)

        accs = lax.fori_loop(
            0, _D, d_body,
            tuple(jnp.zeros((16,), jnp.float32) for _ in range(_G)))
        for g in range(_G):
            outv[pl.ds(g * 16, 16)] = accs[g]
        pltpu.sync_copy(outv, out_hbm.at[pl.ds(e_base + lbase, _C)])

    issue(0, srows0, drows0, sem_s0, sem_d0)

    def pair_body(i, carry):
        c = i * 2
        issue(c + 1, srows1, drows1, sem_s1, sem_d1)
        compute(c, srows0, drows0, sem_s0, sem_d0)

        @pl.when(c + 2 < n_chunks)
        def _():
            issue(c + 2, srows0, drows0, sem_s0, sem_d0)

        compute(c + 1, srows1, drows1, sem_s1, sem_d1)
        return carry

    lax.fori_loop(0, n_chunks // 2, pair_body, 0)


def kernel(z, edge_index, edge_type, rel_emb):
    src = edge_index[0].astype(jnp.int32)
    dst = edge_index[1].astype(jnp.int32)
    et = edge_type.astype(jnp.int32)
    e = src.shape[0]
    mesh = plsc.VectorSubcoreMesh(core_axis_name="c", subcore_axis_name="s")
    f = pl.kernel(
        _distmult_body,
        out_type=jax.ShapeDtypeStruct((e,), jnp.float32),
        mesh=mesh,
        compiler_params=pltpu.CompilerParams(needs_layout_passes=False),
        scratch_types=[
            pltpu.VMEM(rel_emb.shape, jnp.float32),
            pltpu.VMEM((e // _NW,), jnp.int32),
            pltpu.VMEM((e // _NW,), jnp.int32),
            pltpu.VMEM((e // _NW,), jnp.int32),
            pltpu.VMEM((_C, _D), jnp.float32),
            pltpu.VMEM((_C, _D), jnp.float32),
            pltpu.VMEM((_C, _D), jnp.float32),
            pltpu.VMEM((_C, _D), jnp.float32),
            pltpu.VMEM((_C,), jnp.float32),
            pltpu.SemaphoreType.DMA,
            pltpu.SemaphoreType.DMA,
            pltpu.SemaphoreType.DMA,
            pltpu.SemaphoreType.DMA,
        ],
    )
    return f(src, dst, et, z, rel_emb)
